# baseline (speedup 1.0000x reference)
import functools

import jax
import jax.numpy as jnp
from jax import lax
from jax.experimental import pallas as pl
from jax.experimental.pallas import tpu as pltpu
from jax.experimental.pallas import tpu_sc as plsc

N_MESH = 10000
N_GRID = 50000
N_EDGES = 320000
D = 128

NC = 2
NS = 16
NW = NC * NS

G_CHUNK = 12800
N_CHUNKS = 4
G_PAD = G_CHUNK * N_CHUNKS
EC = 80

_HI = lax.Precision.HIGHEST


def _matmul_body(x_ref, w_ref, o_ref):
    o_ref[...] = jnp.dot(x_ref[...], w_ref[...],
                         preferred_element_type=jnp.float32, precision=_HI)


def _rowblock_matmul(x, w, bm):
    m, k = x.shape
    n = w.shape[1]
    return pl.pallas_call(
        _matmul_body,
        grid=(m // bm,),
        in_specs=[pl.BlockSpec((bm, k), lambda i: (i, 0)),
                  pl.BlockSpec((k, n), lambda i: (0, 0))],
        out_specs=pl.BlockSpec((bm, n), lambda i: (i, 0)),
        out_shape=jax.ShapeDtypeStruct((m, n), jnp.float32),
    )(x, w)


def _edge_mlp_body(x_ref, g_ref, w1_ref, b1_ref, w2_ref, b2_ref,
                   gm_ref, bt_ref, o_ref):
    h = jnp.dot(x_ref[...], w1_ref[...],
                preferred_element_type=jnp.float32, precision=_HI)
    h = h + g_ref[...] + b1_ref[...]
    h = h * jax.nn.sigmoid(h)
    h = jnp.dot(h, w2_ref[...],
                preferred_element_type=jnp.float32, precision=_HI) + b2_ref[...]
    mu = jnp.mean(h, axis=-1, keepdims=True)
    var = jnp.mean((h - mu) ** 2, axis=-1, keepdims=True)
    h = (h - mu) * lax.rsqrt(var + 1e-5)
    o_ref[...] = h * gm_ref[...] + bt_ref[...]


def _edge_mlp(m2g, gsum, w1a, b1, w2, b2, gm, bt, bm):
    m = m2g.shape[0]
    vec = lambda i: (0, 0)
    return pl.pallas_call(
        _edge_mlp_body,
        grid=(m // bm,),
        in_specs=[pl.BlockSpec((bm, D), lambda i: (i, 0)),
                  pl.BlockSpec((bm, D), lambda i: (i, 0)),
                  pl.BlockSpec((D, D), vec),
                  pl.BlockSpec((1, D), vec),
                  pl.BlockSpec((D, D), vec),
                  pl.BlockSpec((1, D), vec),
                  pl.BlockSpec((1, D), vec),
                  pl.BlockSpec((1, D), vec)],
        out_specs=pl.BlockSpec((bm, D), lambda i: (i, 0)),
        out_shape=jax.ShapeDtypeStruct((m, D), jnp.float32),
    )(m2g, gsum, w1a, b1, w2, b2, gm, bt)


def _node_mlp_body(a_ref, gn_ref, wa_ref, wg_ref, b1_ref, w2_ref, b2_ref,
                   gm_ref, bt_ref, o_ref):
    h = jnp.dot(a_ref[...], wa_ref[...],
                preferred_element_type=jnp.float32, precision=_HI)
    h = h + jnp.dot(gn_ref[...], wg_ref[...],
                    preferred_element_type=jnp.float32, precision=_HI)
    h = h + b1_ref[...]
    h = h * jax.nn.sigmoid(h)
    h = jnp.dot(h, w2_ref[...],
                preferred_element_type=jnp.float32, precision=_HI) + b2_ref[...]
    mu = jnp.mean(h, axis=-1, keepdims=True)
    var = jnp.mean((h - mu) ** 2, axis=-1, keepdims=True)
    h = (h - mu) * lax.rsqrt(var + 1e-5)
    o_ref[...] = h * gm_ref[...] + bt_ref[...] + gn_ref[...]


def _node_mlp(agg_pad, gn, wa, wg, b1, w2, b2, gm, bt, bm):
    vec = lambda i: (0, 0)
    return pl.pallas_call(
        _node_mlp_body,
        grid=(N_GRID // bm,),
        in_specs=[pl.BlockSpec((bm, D), lambda i: (i, 0)),
                  pl.BlockSpec((bm, D), lambda i: (i, 0)),
                  pl.BlockSpec((D, D), vec),
                  pl.BlockSpec((D, D), vec),
                  pl.BlockSpec((1, D), vec),
                  pl.BlockSpec((D, D), vec),
                  pl.BlockSpec((1, D), vec),
                  pl.BlockSpec((1, D), vec),
                  pl.BlockSpec((1, D), vec)],
        out_specs=pl.BlockSpec((bm, D), lambda i: (i, 0)),
        out_shape=jax.ShapeDtypeStruct((N_GRID, D), jnp.float32),
    )(agg_pad, gn, wa, wg, b1, w2, b2, gm, bt)


GW = 80
NWIN = N_GRID // GW
ECK = 512
NTH = 640


def _rowptr_body(d_ref, o_ref):
    i = pl.program_id(0)

    @pl.when(i == 0)
    def _():
        o_ref[...] = jnp.zeros_like(o_ref)

    d = d_ref[...].reshape(-1, 1)
    th = lax.broadcasted_iota(jnp.int32, (1, NTH), 1) * GW
    cnt = jnp.sum((d < th).astype(jnp.int32), axis=0, keepdims=True)
    o_ref[...] += cnt


def _tc_rowptr(dst):
    d2 = dst.reshape(N_EDGES // ECK, 1, ECK)
    return pl.pallas_call(
        _rowptr_body,
        grid=(N_EDGES // ECK,),
        in_specs=[pl.BlockSpec((1, 1, ECK), lambda i: (i, 0, 0))],
        out_specs=pl.BlockSpec((1, NTH), lambda i: (0, 0)),
        out_shape=jax.ShapeDtypeStruct((1, NTH), jnp.int32),
    )(d2)


def _segsum_body(rp_ref, ef_hbm, d_hbm, o_ref, ebuf, dbuf, sem1, sem2):
    g = pl.program_id(0)
    lo = rp_ref[0, g]
    hi = rp_ref[0, g + 1]
    gbase = g * GW
    wio = lax.broadcasted_iota(jnp.int32, (GW, 128), 0)
    lio = lax.broadcasted_iota(jnp.int32, (1, 128), 1)
    pos0 = (lo // 128) * 128

    def cond(state):
        return state[0] < hi

    def body(state):
        pos, acc = state
        pos_r = jnp.minimum(pos, N_EDGES - ECK)
        cp1 = pltpu.make_async_copy(ef_hbm.at[pl.ds(pos_r, ECK)], ebuf, sem1)
        cp2 = pltpu.make_async_copy(
            d_hbm.at[pl.ds(pos_r // 128, ECK // 128)], dbuf, sem2)
        cp1.start()
        cp2.start()
        cp1.wait()
        cp2.wait()
        for h in range(ECK // 128):
            dv = dbuf[h, :].reshape(1, 128)
            ev = lio + (pos_r + h * 128)
            valid = (dv - gbase == wio) & (ev >= pos)
            oh = valid.astype(jnp.float32)
            acc = acc + jnp.dot(oh, ebuf[pl.ds(h * 128, 128), :],
                                preferred_element_type=jnp.float32,
                                precision=_HI)
        return (pos_r + ECK, acc)

    _, acc = lax.while_loop(
        cond, body, (pos0, jnp.zeros((GW, D), jnp.float32)))
    o_ref[...] = acc


def _tc_segment_sum(efeat, dst, rowptr):
    d2 = dst.reshape(N_EDGES // 128, 128)
    return pl.pallas_call(
        _segsum_body,
        grid=(NWIN,),
        in_specs=[pl.BlockSpec(memory_space=pltpu.SMEM),
                  pl.BlockSpec(memory_space=pltpu.HBM),
                  pl.BlockSpec(memory_space=pltpu.HBM)],
        out_specs=pl.BlockSpec((GW, D), lambda g: (g, 0)),
        out_shape=jax.ShapeDtypeStruct((N_GRID, D), jnp.float32),
        scratch_shapes=[pltpu.VMEM((ECK, D), jnp.float32),
                        pltpu.VMEM((ECK // 128, 128), jnp.int32),
                        pltpu.SemaphoreType.DMA,
                        pltpu.SemaphoreType.DMA],
    )(rowptr, efeat, d2)


def _sc_gather_sum(meshp, gridp, src, dst):
    n_per_w = N_EDGES // NW
    mesh_sc = plsc.VectorSubcoreMesh(core_axis_name="c", subcore_axis_name="s")

    @functools.partial(
        pl.kernel,
        mesh=mesh_sc,
        out_type=jax.ShapeDtypeStruct((N_EDGES, D), jnp.float32),
        scratch_types=[
            pltpu.VMEM((EC,), jnp.int32),
            pltpu.VMEM((EC,), jnp.int32),
            pltpu.VMEM((EC, D), jnp.float32),
            pltpu.VMEM((EC, D), jnp.float32),
            pltpu.SemaphoreType.DMA,
            pltpu.SemaphoreType.DMA,
        ],
    )
    def k(meshp_hbm, gridp_hbm, src_hbm, dst_hbm, out_hbm,
          idx1, idx2, rows1, rows2, sem1, sem2):
        wid = lax.axis_index("s") * NC + lax.axis_index("c")
        base = wid * n_per_w

        def chunk(ci, carry):
            pos = base + ci * EC
            pltpu.sync_copy(src_hbm.at[pl.ds(pos, EC)], idx1)
            pltpu.sync_copy(dst_hbm.at[pl.ds(pos, EC)], idx2)
            c1 = pltpu.async_copy(meshp_hbm.at[idx1], rows1, sem1)
            c2 = pltpu.async_copy(gridp_hbm.at[idx2], rows2, sem2)
            c1.wait()
            c2.wait()

            def addrow(r, c):
                for v in range(D // 16):
                    sl = pl.ds(v * 16, 16)
                    rows1[r, sl] = rows1[r, sl] + rows2[r, sl]
                return c

            lax.fori_loop(0, EC, addrow, 0)
            pltpu.sync_copy(rows1, out_hbm.at[pl.ds(pos, EC)])
            return carry

        lax.fori_loop(0, n_per_w // EC, chunk, 0)

    return k(meshp, gridp, src, dst)


def kernel(m2g_efeat, grid_nfeat, mesh_nfeat, src, dst,
           eW1, eb1, eW2, eb2, eg, ebeta,
           nW1, nb1, nW2, nb2, ng, nbeta):
    src = src.astype(jnp.int32)
    dst = dst.astype(jnp.int32)
    w_e = eW1[:D]
    w_m = eW1[D:2 * D]
    w_g = eW1[2 * D:]
    mesh_proj = _rowblock_matmul(mesh_nfeat, w_m, bm=400)
    grid_proj = _rowblock_matmul(grid_nfeat, w_g, bm=1000)
    gsum = _sc_gather_sum(mesh_proj, grid_proj, src, dst)
    efeat = _edge_mlp(m2g_efeat, gsum, w_e,
                      eb1.reshape(1, D), eW2, eb2.reshape(1, D),
                      eg.reshape(1, D), ebeta.reshape(1, D), bm=512)
    rowptr = _tc_rowptr(dst)
    agg = _tc_segment_sum(efeat, dst, rowptr)
    out = _node_mlp(agg, grid_nfeat, nW1[:D], nW1[D:],
                    nb1.reshape(1, D), nW2, nb2.reshape(1, D),
                    ng.reshape(1, D), nbeta.reshape(1, D), bm=1000)
    return out


# speedup vs baseline: 1.3590x; 1.3590x over previous
import functools

import jax
import jax.numpy as jnp
from jax import lax
from jax.experimental import pallas as pl
from jax.experimental.pallas import tpu as pltpu
from jax.experimental.pallas import tpu_sc as plsc

N_MESH = 10000
N_GRID = 50000
N_EDGES = 320000
D = 128

NC = 2
NS = 16
NW = NC * NS

G_CHUNK = 12800
N_CHUNKS = 4
G_PAD = G_CHUNK * N_CHUNKS
EC = 80

_HI = lax.Precision.HIGHEST


def _matmul_body(x_ref, w_ref, o_ref):
    o_ref[...] = jnp.dot(x_ref[...], w_ref[...],
                         preferred_element_type=jnp.float32, precision=_HI)


def _rowblock_matmul(x, w, bm):
    m, k = x.shape
    n = w.shape[1]
    return pl.pallas_call(
        _matmul_body,
        grid=(m // bm,),
        in_specs=[pl.BlockSpec((bm, k), lambda i: (i, 0)),
                  pl.BlockSpec((k, n), lambda i: (0, 0))],
        out_specs=pl.BlockSpec((bm, n), lambda i: (i, 0)),
        out_shape=jax.ShapeDtypeStruct((m, n), jnp.float32),
    )(x, w)


def _edge_mlp_body(x_ref, g_ref, w1_ref, b1_ref, w2_ref, b2_ref,
                   gm_ref, bt_ref, o_ref):
    h = jnp.dot(x_ref[...], w1_ref[...],
                preferred_element_type=jnp.float32, precision=_HI)
    h = h + g_ref[...] + b1_ref[...]
    h = h * jax.nn.sigmoid(h)
    h = jnp.dot(h, w2_ref[...],
                preferred_element_type=jnp.float32, precision=_HI) + b2_ref[...]
    mu = jnp.mean(h, axis=-1, keepdims=True)
    var = jnp.mean((h - mu) ** 2, axis=-1, keepdims=True)
    h = (h - mu) * lax.rsqrt(var + 1e-5)
    o_ref[...] = h * gm_ref[...] + bt_ref[...]


def _edge_mlp(m2g, gsum, w1a, b1, w2, b2, gm, bt, bm):
    m = m2g.shape[0]
    vec = lambda i: (0, 0)
    return pl.pallas_call(
        _edge_mlp_body,
        grid=(m // bm,),
        in_specs=[pl.BlockSpec((bm, D), lambda i: (i, 0)),
                  pl.BlockSpec((bm, D), lambda i: (i, 0)),
                  pl.BlockSpec((D, D), vec),
                  pl.BlockSpec((1, D), vec),
                  pl.BlockSpec((D, D), vec),
                  pl.BlockSpec((1, D), vec),
                  pl.BlockSpec((1, D), vec),
                  pl.BlockSpec((1, D), vec)],
        out_specs=pl.BlockSpec((bm, D), lambda i: (i, 0)),
        out_shape=jax.ShapeDtypeStruct((m, D), jnp.float32),
    )(m2g, gsum, w1a, b1, w2, b2, gm, bt)


def _node_mlp_body(a_ref, gn_ref, wa_ref, wg_ref, b1_ref, w2_ref, b2_ref,
                   gm_ref, bt_ref, o_ref):
    h = jnp.dot(a_ref[...], wa_ref[...],
                preferred_element_type=jnp.float32, precision=_HI)
    h = h + jnp.dot(gn_ref[...], wg_ref[...],
                    preferred_element_type=jnp.float32, precision=_HI)
    h = h + b1_ref[...]
    h = h * jax.nn.sigmoid(h)
    h = jnp.dot(h, w2_ref[...],
                preferred_element_type=jnp.float32, precision=_HI) + b2_ref[...]
    mu = jnp.mean(h, axis=-1, keepdims=True)
    var = jnp.mean((h - mu) ** 2, axis=-1, keepdims=True)
    h = (h - mu) * lax.rsqrt(var + 1e-5)
    o_ref[...] = h * gm_ref[...] + bt_ref[...] + gn_ref[...]


def _node_mlp(agg_pad, gn, wa, wg, b1, w2, b2, gm, bt, bm):
    vec = lambda i: (0, 0)
    return pl.pallas_call(
        _node_mlp_body,
        grid=(N_GRID // bm,),
        in_specs=[pl.BlockSpec((bm, D), lambda i: (i, 0)),
                  pl.BlockSpec((bm, D), lambda i: (i, 0)),
                  pl.BlockSpec((D, D), vec),
                  pl.BlockSpec((D, D), vec),
                  pl.BlockSpec((1, D), vec),
                  pl.BlockSpec((D, D), vec),
                  pl.BlockSpec((1, D), vec),
                  pl.BlockSpec((1, D), vec),
                  pl.BlockSpec((1, D), vec)],
        out_specs=pl.BlockSpec((bm, D), lambda i: (i, 0)),
        out_shape=jax.ShapeDtypeStruct((N_GRID, D), jnp.float32),
    )(agg_pad, gn, wa, wg, b1, w2, b2, gm, bt)


GW = 400
NWIN = N_GRID // GW
ECK = 1024
NTH = 640
RPB = 512


def _rowptr_body(d_ref, o_ref):
    i = pl.program_id(0)

    @pl.when(i == 0)
    def _():
        o_ref[...] = jnp.zeros_like(o_ref)

    d = d_ref[...].reshape(-1, 1)
    th = lax.broadcasted_iota(jnp.int32, (1, NTH), 1) * GW
    cnt = jnp.sum((d < th).astype(jnp.int32), axis=0, keepdims=True)
    o_ref[...] += cnt


def _tc_rowptr(dst):
    d2 = dst.reshape(N_EDGES // RPB, 1, RPB)
    return pl.pallas_call(
        _rowptr_body,
        grid=(N_EDGES // RPB,),
        in_specs=[pl.BlockSpec((1, 1, RPB), lambda i: (i, 0, 0))],
        out_specs=pl.BlockSpec((1, NTH), lambda i: (0, 0)),
        out_shape=jax.ShapeDtypeStruct((1, NTH), jnp.int32),
    )(d2)


def _segsum_body(rp_ref, ef_hbm, d_hbm, o_ref, acc,
                 ebuf0, ebuf1, dbuf0, dbuf1, es0, es1, ds0, ds1):
    g = pl.program_id(0)
    lo = rp_ref[0, g]
    hi = rp_ref[0, g + 1]
    gbase = g * GW
    wio = lax.broadcasted_iota(jnp.int32, (GW, 128), 0)
    lio = lax.broadcasted_iota(jnp.int32, (1, 128), 1)
    pos0 = (lo // 128) * 128
    acc[...] = jnp.zeros((GW, D), jnp.float32)
    ebufs, dbufs = (ebuf0, ebuf1), (dbuf0, dbuf1)
    esems, dsems = (es0, es1), (ds0, ds1)

    def start(pos, b):
        pos_r = jnp.minimum(pos, N_EDGES - ECK)
        pltpu.make_async_copy(
            ef_hbm.at[pl.ds(pos_r, ECK)], ebufs[b], esems[b]).start()
        pltpu.make_async_copy(
            d_hbm.at[pl.ds(pos_r // 128, ECK // 128)], dbufs[b], dsems[b]).start()

    @pl.when(pos0 < hi)
    def _():
        start(pos0, 0)

    def cond(state):
        return state[0] < hi

    def body(state):
        pos, it = state
        pos_r = jnp.minimum(pos, N_EDGES - ECK)
        nxt = pos_r + ECK

        def process(b):
            pltpu.make_async_copy(
                ef_hbm.at[pl.ds(pos_r, ECK)], ebufs[b], esems[b]).wait()
            pltpu.make_async_copy(
                d_hbm.at[pl.ds(pos_r // 128, ECK // 128)], dbufs[b],
                dsems[b]).wait()

            @pl.when(nxt < hi)
            def _():
                start(nxt, 1 - b)

            for h in range(ECK // 128):
                dv = dbufs[b][h, :].reshape(1, 128)
                ev = lio + (pos_r + h * 128)
                valid = (dv - gbase == wio) & (ev >= pos)
                oh = valid.astype(jnp.float32)
                acc[...] += jnp.dot(oh, ebufs[b][pl.ds(h * 128, 128), :],
                                    preferred_element_type=jnp.float32,
                                    precision=_HI)

        @pl.when(it % 2 == 0)
        def _():
            process(0)

        @pl.when(it % 2 == 1)
        def _():
            process(1)

        return (nxt, it + 1)

    lax.while_loop(cond, body, (pos0, 0))
    o_ref[...] = acc[...]


def _tc_segment_sum(efeat, dst, rowptr):
    d2 = dst.reshape(N_EDGES // 128, 128)
    return pl.pallas_call(
        _segsum_body,
        grid=(NWIN,),
        in_specs=[pl.BlockSpec(memory_space=pltpu.SMEM),
                  pl.BlockSpec(memory_space=pltpu.HBM),
                  pl.BlockSpec(memory_space=pltpu.HBM)],
        out_specs=pl.BlockSpec((GW, D), lambda g: (g, 0)),
        out_shape=jax.ShapeDtypeStruct((N_GRID, D), jnp.float32),
        scratch_shapes=[pltpu.VMEM((GW, D), jnp.float32),
                        pltpu.VMEM((ECK, D), jnp.float32),
                        pltpu.VMEM((ECK, D), jnp.float32),
                        pltpu.VMEM((ECK // 128, 128), jnp.int32),
                        pltpu.VMEM((ECK // 128, 128), jnp.int32),
                        pltpu.SemaphoreType.DMA,
                        pltpu.SemaphoreType.DMA,
                        pltpu.SemaphoreType.DMA,
                        pltpu.SemaphoreType.DMA],
    )(rowptr, efeat, d2)


def _sc_gather_sum(meshp, gridp, src, dst):
    n_per_w = N_EDGES // NW
    mesh_sc = plsc.VectorSubcoreMesh(core_axis_name="c", subcore_axis_name="s")

    @functools.partial(
        pl.kernel,
        mesh=mesh_sc,
        out_type=jax.ShapeDtypeStruct((N_EDGES, D), jnp.float32),
        scratch_types=[
            pltpu.VMEM((EC,), jnp.int32),
            pltpu.VMEM((EC,), jnp.int32),
            pltpu.VMEM((EC, D), jnp.float32),
            pltpu.VMEM((EC, D), jnp.float32),
            pltpu.SemaphoreType.DMA,
            pltpu.SemaphoreType.DMA,
        ],
    )
    def k(meshp_hbm, gridp_hbm, src_hbm, dst_hbm, out_hbm,
          idx1, idx2, rows1, rows2, sem1, sem2):
        wid = lax.axis_index("s") * NC + lax.axis_index("c")
        base = wid * n_per_w

        def chunk(ci, carry):
            pos = base + ci * EC
            pltpu.sync_copy(src_hbm.at[pl.ds(pos, EC)], idx1)
            pltpu.sync_copy(dst_hbm.at[pl.ds(pos, EC)], idx2)
            c1 = pltpu.async_copy(meshp_hbm.at[idx1], rows1, sem1)
            c2 = pltpu.async_copy(gridp_hbm.at[idx2], rows2, sem2)
            c1.wait()
            c2.wait()

            def addrow(r, c):
                for v in range(D // 16):
                    sl = pl.ds(v * 16, 16)
                    rows1[r, sl] = rows1[r, sl] + rows2[r, sl]
                return c

            lax.fori_loop(0, EC, addrow, 0)
            pltpu.sync_copy(rows1, out_hbm.at[pl.ds(pos, EC)])
            return carry

        lax.fori_loop(0, n_per_w // EC, chunk, 0)

    return k(meshp, gridp, src, dst)


def kernel(m2g_efeat, grid_nfeat, mesh_nfeat, src, dst,
           eW1, eb1, eW2, eb2, eg, ebeta,
           nW1, nb1, nW2, nb2, ng, nbeta):
    src = src.astype(jnp.int32)
    dst = dst.astype(jnp.int32)
    w_e = eW1[:D]
    w_m = eW1[D:2 * D]
    w_g = eW1[2 * D:]
    mesh_proj = _rowblock_matmul(mesh_nfeat, w_m, bm=400)
    grid_proj = _rowblock_matmul(grid_nfeat, w_g, bm=1000)
    gsum = _sc_gather_sum(mesh_proj, grid_proj, src, dst)
    efeat = _edge_mlp(m2g_efeat, gsum, w_e,
                      eb1.reshape(1, D), eW2, eb2.reshape(1, D),
                      eg.reshape(1, D), ebeta.reshape(1, D), bm=512)
    rowptr = _tc_rowptr(dst)
    agg = _tc_segment_sum(efeat, dst, rowptr)
    out = _node_mlp(agg, grid_nfeat, nW1[:D], nW1[D:],
                    nb1.reshape(1, D), nW2, nb2.reshape(1, D),
                    ng.reshape(1, D), nbeta.reshape(1, D), bm=1000)
    return out


# speedup vs baseline: 1.4494x; 1.0665x over previous
import functools

import jax
import jax.numpy as jnp
from jax import lax
from jax.experimental import pallas as pl
from jax.experimental.pallas import tpu as pltpu
from jax.experimental.pallas import tpu_sc as plsc

N_MESH = 10000
N_GRID = 50000
N_EDGES = 320000
D = 128

NC = 2
NS = 16
NW = NC * NS

G_CHUNK = 12800
N_CHUNKS = 4
G_PAD = G_CHUNK * N_CHUNKS
EC = 80

_HI = lax.Precision.HIGHEST


def _matmul_body(x_ref, w_ref, o_ref):
    o_ref[...] = jnp.dot(x_ref[...], w_ref[...],
                         preferred_element_type=jnp.float32, precision=_HI)


def _rowblock_matmul(x, w, bm):
    m, k = x.shape
    n = w.shape[1]
    return pl.pallas_call(
        _matmul_body,
        grid=(m // bm,),
        in_specs=[pl.BlockSpec((bm, k), lambda i: (i, 0)),
                  pl.BlockSpec((k, n), lambda i: (0, 0))],
        out_specs=pl.BlockSpec((bm, n), lambda i: (i, 0)),
        out_shape=jax.ShapeDtypeStruct((m, n), jnp.float32),
    )(x, w)


def _edge_mlp_body(x_ref, g_ref, w1_ref, b1_ref, w2_ref, b2_ref,
                   gm_ref, bt_ref, o_ref):
    h = jnp.dot(x_ref[...], w1_ref[...],
                preferred_element_type=jnp.float32, precision=_HI)
    h = h + g_ref[...] + b1_ref[...]
    h = h * jax.nn.sigmoid(h)
    h = jnp.dot(h, w2_ref[...],
                preferred_element_type=jnp.float32, precision=_HI) + b2_ref[...]
    mu = jnp.mean(h, axis=-1, keepdims=True)
    var = jnp.mean((h - mu) ** 2, axis=-1, keepdims=True)
    h = (h - mu) * lax.rsqrt(var + 1e-5)
    o_ref[...] = h * gm_ref[...] + bt_ref[...]


def _edge_mlp(m2g, gsum, w1a, b1, w2, b2, gm, bt, bm):
    m = m2g.shape[0]
    vec = lambda i: (0, 0)
    return pl.pallas_call(
        _edge_mlp_body,
        grid=(m // bm,),
        in_specs=[pl.BlockSpec((bm, D), lambda i: (i, 0)),
                  pl.BlockSpec((bm, D), lambda i: (i, 0)),
                  pl.BlockSpec((D, D), vec),
                  pl.BlockSpec((1, D), vec),
                  pl.BlockSpec((D, D), vec),
                  pl.BlockSpec((1, D), vec),
                  pl.BlockSpec((1, D), vec),
                  pl.BlockSpec((1, D), vec)],
        out_specs=pl.BlockSpec((bm, D), lambda i: (i, 0)),
        out_shape=jax.ShapeDtypeStruct((m, D), jnp.float32),
    )(m2g, gsum, w1a, b1, w2, b2, gm, bt)


def _node_mlp_body(a_ref, gn_ref, wa_ref, wg_ref, b1_ref, w2_ref, b2_ref,
                   gm_ref, bt_ref, o_ref):
    h = jnp.dot(a_ref[...], wa_ref[...],
                preferred_element_type=jnp.float32, precision=_HI)
    h = h + jnp.dot(gn_ref[...], wg_ref[...],
                    preferred_element_type=jnp.float32, precision=_HI)
    h = h + b1_ref[...]
    h = h * jax.nn.sigmoid(h)
    h = jnp.dot(h, w2_ref[...],
                preferred_element_type=jnp.float32, precision=_HI) + b2_ref[...]
    mu = jnp.mean(h, axis=-1, keepdims=True)
    var = jnp.mean((h - mu) ** 2, axis=-1, keepdims=True)
    h = (h - mu) * lax.rsqrt(var + 1e-5)
    o_ref[...] = h * gm_ref[...] + bt_ref[...] + gn_ref[...]


def _node_mlp(agg_pad, gn, wa, wg, b1, w2, b2, gm, bt, bm):
    vec = lambda i: (0, 0)
    return pl.pallas_call(
        _node_mlp_body,
        grid=(N_GRID // bm,),
        in_specs=[pl.BlockSpec((bm, D), lambda i: (i, 0)),
                  pl.BlockSpec((bm, D), lambda i: (i, 0)),
                  pl.BlockSpec((D, D), vec),
                  pl.BlockSpec((D, D), vec),
                  pl.BlockSpec((1, D), vec),
                  pl.BlockSpec((D, D), vec),
                  pl.BlockSpec((1, D), vec),
                  pl.BlockSpec((1, D), vec),
                  pl.BlockSpec((1, D), vec)],
        out_specs=pl.BlockSpec((bm, D), lambda i: (i, 0)),
        out_shape=jax.ShapeDtypeStruct((N_GRID, D), jnp.float32),
    )(agg_pad, gn, wa, wg, b1, w2, b2, gm, bt)


GW = 200
NWIN = N_GRID // GW
ECK = 1024
NTH = 640
RPB = 512


def _rowptr_body(d_ref, o_ref):
    i = pl.program_id(0)

    @pl.when(i == 0)
    def _():
        o_ref[...] = jnp.zeros_like(o_ref)

    d = d_ref[...].reshape(-1, 1)
    th = lax.broadcasted_iota(jnp.int32, (1, NTH), 1) * GW
    cnt = jnp.sum((d < th).astype(jnp.int32), axis=0, keepdims=True)
    o_ref[...] += cnt


def _tc_rowptr(dst):
    d2 = dst.reshape(N_EDGES // RPB, 1, RPB)
    return pl.pallas_call(
        _rowptr_body,
        grid=(N_EDGES // RPB,),
        in_specs=[pl.BlockSpec((1, 1, RPB), lambda i: (i, 0, 0))],
        out_specs=pl.BlockSpec((1, NTH), lambda i: (0, 0)),
        out_shape=jax.ShapeDtypeStruct((1, NTH), jnp.int32),
    )(d2)


def _segsum_body(rp_ref, ef_hbm, d_hbm, o_ref, acc,
                 ebuf0, ebuf1, dbuf0, dbuf1, es0, es1, ds0, ds1):
    g = pl.program_id(0)
    lo = rp_ref[0, g]
    hi = rp_ref[0, g + 1]
    gbase = g * GW
    wio = lax.broadcasted_iota(jnp.int32, (GW, 128), 0)
    lio = lax.broadcasted_iota(jnp.int32, (1, 128), 1)
    pos0 = (lo // 128) * 128
    acc[...] = jnp.zeros((GW, D), jnp.float32)
    ebufs, dbufs = (ebuf0, ebuf1), (dbuf0, dbuf1)
    esems, dsems = (es0, es1), (ds0, ds1)

    def start(pos, b):
        pos_r = jnp.minimum(pos, N_EDGES - ECK)
        pltpu.make_async_copy(
            ef_hbm.at[pl.ds(pos_r, ECK)], ebufs[b], esems[b]).start()
        pltpu.make_async_copy(
            d_hbm.at[pl.ds(pos_r // 128, ECK // 128)], dbufs[b], dsems[b]).start()

    @pl.when(pos0 < hi)
    def _():
        start(pos0, 0)

    def cond(state):
        return state[0] < hi

    def body(state):
        pos, it = state
        pos_r = jnp.minimum(pos, N_EDGES - ECK)
        nxt = pos_r + ECK

        def process(b):
            pltpu.make_async_copy(
                ef_hbm.at[pl.ds(pos_r, ECK)], ebufs[b], esems[b]).wait()
            pltpu.make_async_copy(
                d_hbm.at[pl.ds(pos_r // 128, ECK // 128)], dbufs[b],
                dsems[b]).wait()

            @pl.when(nxt < hi)
            def _():
                start(nxt, 1 - b)

            for h in range(ECK // 128):
                dv = dbufs[b][h, :].reshape(1, 128)
                ev = lio + (pos_r + h * 128)
                valid = (dv - gbase == wio) & (ev >= pos)
                oh = valid.astype(jnp.float32)
                acc[...] += jnp.dot(oh, ebufs[b][pl.ds(h * 128, 128), :],
                                    preferred_element_type=jnp.float32)

        @pl.when(it % 2 == 0)
        def _():
            process(0)

        @pl.when(it % 2 == 1)
        def _():
            process(1)

        return (nxt, it + 1)

    lax.while_loop(cond, body, (pos0, 0))
    o_ref[...] = acc[...]


def _tc_segment_sum(efeat, dst, rowptr):
    d2 = dst.reshape(N_EDGES // 128, 128)
    return pl.pallas_call(
        _segsum_body,
        grid=(NWIN,),
        in_specs=[pl.BlockSpec(memory_space=pltpu.SMEM),
                  pl.BlockSpec(memory_space=pltpu.HBM),
                  pl.BlockSpec(memory_space=pltpu.HBM)],
        out_specs=pl.BlockSpec((GW, D), lambda g: (g, 0)),
        out_shape=jax.ShapeDtypeStruct((N_GRID, D), jnp.float32),
        scratch_shapes=[pltpu.VMEM((GW, D), jnp.float32),
                        pltpu.VMEM((ECK, D), jnp.float32),
                        pltpu.VMEM((ECK, D), jnp.float32),
                        pltpu.VMEM((ECK // 128, 128), jnp.int32),
                        pltpu.VMEM((ECK // 128, 128), jnp.int32),
                        pltpu.SemaphoreType.DMA,
                        pltpu.SemaphoreType.DMA,
                        pltpu.SemaphoreType.DMA,
                        pltpu.SemaphoreType.DMA],
    )(rowptr, efeat, d2)


def _sc_gather_sum(meshp, gridp, src, dst):
    n_per_w = N_EDGES // NW
    mesh_sc = plsc.VectorSubcoreMesh(core_axis_name="c", subcore_axis_name="s")

    @functools.partial(
        pl.kernel,
        mesh=mesh_sc,
        out_type=jax.ShapeDtypeStruct((N_EDGES, D), jnp.float32),
        scratch_types=[
            pltpu.VMEM((EC,), jnp.int32),
            pltpu.VMEM((EC,), jnp.int32),
            pltpu.VMEM((EC, D), jnp.float32),
            pltpu.VMEM((EC, D), jnp.float32),
            pltpu.SemaphoreType.DMA,
            pltpu.SemaphoreType.DMA,
        ],
    )
    def k(meshp_hbm, gridp_hbm, src_hbm, dst_hbm, out_hbm,
          idx1, idx2, rows1, rows2, sem1, sem2):
        wid = lax.axis_index("s") * NC + lax.axis_index("c")
        base = wid * n_per_w

        def chunk(ci, carry):
            pos = base + ci * EC
            pltpu.sync_copy(src_hbm.at[pl.ds(pos, EC)], idx1)
            pltpu.sync_copy(dst_hbm.at[pl.ds(pos, EC)], idx2)
            c1 = pltpu.async_copy(meshp_hbm.at[idx1], rows1, sem1)
            c2 = pltpu.async_copy(gridp_hbm.at[idx2], rows2, sem2)
            c1.wait()
            c2.wait()

            def addrow(r, c):
                for v in range(D // 16):
                    sl = pl.ds(v * 16, 16)
                    rows1[r, sl] = rows1[r, sl] + rows2[r, sl]
                return c

            lax.fori_loop(0, EC, addrow, 0)
            pltpu.sync_copy(rows1, out_hbm.at[pl.ds(pos, EC)])
            return carry

        lax.fori_loop(0, n_per_w // EC, chunk, 0)

    return k(meshp, gridp, src, dst)


def kernel(m2g_efeat, grid_nfeat, mesh_nfeat, src, dst,
           eW1, eb1, eW2, eb2, eg, ebeta,
           nW1, nb1, nW2, nb2, ng, nbeta):
    src = src.astype(jnp.int32)
    dst = dst.astype(jnp.int32)
    w_e = eW1[:D]
    w_m = eW1[D:2 * D]
    w_g = eW1[2 * D:]
    mesh_proj = _rowblock_matmul(mesh_nfeat, w_m, bm=400)
    grid_proj = _rowblock_matmul(grid_nfeat, w_g, bm=1000)
    gsum = _sc_gather_sum(mesh_proj, grid_proj, src, dst)
    efeat = _edge_mlp(m2g_efeat, gsum, w_e,
                      eb1.reshape(1, D), eW2, eb2.reshape(1, D),
                      eg.reshape(1, D), ebeta.reshape(1, D), bm=512)
    rowptr = _tc_rowptr(dst)
    agg = _tc_segment_sum(efeat, dst, rowptr)
    out = _node_mlp(agg, grid_nfeat, nW1[:D], nW1[D:],
                    nb1.reshape(1, D), nW2, nb2.reshape(1, D),
                    ng.reshape(1, D), nbeta.reshape(1, D), bm=1000)
    return out


# speedup vs baseline: 1.5584x; 1.0753x over previous
import functools

import jax
import jax.numpy as jnp
from jax import lax
from jax.experimental import pallas as pl
from jax.experimental.pallas import tpu as pltpu
from jax.experimental.pallas import tpu_sc as plsc

N_MESH = 10000
N_GRID = 50000
N_EDGES = 320000
D = 128

NC = 2
NS = 16
NW = NC * NS

G_CHUNK = 12800
N_CHUNKS = 4
G_PAD = G_CHUNK * N_CHUNKS
EC = 80

_HI = lax.Precision.HIGHEST


def _matmul_body(x_ref, w_ref, o_ref):
    o_ref[...] = jnp.dot(x_ref[...], w_ref[...],
                         preferred_element_type=jnp.float32, precision=_HI)


def _rowblock_matmul(x, w, bm):
    m, k = x.shape
    n = w.shape[1]
    return pl.pallas_call(
        _matmul_body,
        grid=(m // bm,),
        in_specs=[pl.BlockSpec((bm, k), lambda i: (i, 0)),
                  pl.BlockSpec((k, n), lambda i: (0, 0))],
        out_specs=pl.BlockSpec((bm, n), lambda i: (i, 0)),
        out_shape=jax.ShapeDtypeStruct((m, n), jnp.float32),
    )(x, w)


def _edge_mlp_body(x_ref, g_ref, w1_ref, b1_ref, w2_ref, b2_ref,
                   gm_ref, bt_ref, o_ref):
    h = jnp.dot(x_ref[...], w1_ref[...],
                preferred_element_type=jnp.float32, precision=_HI)
    h = h + g_ref[...] + b1_ref[...]
    h = h * jax.nn.sigmoid(h)
    h = jnp.dot(h, w2_ref[...],
                preferred_element_type=jnp.float32, precision=_HI) + b2_ref[...]
    mu = jnp.mean(h, axis=-1, keepdims=True)
    var = jnp.mean((h - mu) ** 2, axis=-1, keepdims=True)
    h = (h - mu) * lax.rsqrt(var + 1e-5)
    o_ref[...] = h * gm_ref[...] + bt_ref[...]


def _edge_mlp(m2g, gsum, w1a, b1, w2, b2, gm, bt, bm):
    m = m2g.shape[0]
    vec = lambda i: (0, 0)
    return pl.pallas_call(
        _edge_mlp_body,
        grid=(m // bm,),
        in_specs=[pl.BlockSpec((bm, D), lambda i: (i, 0)),
                  pl.BlockSpec((bm, D), lambda i: (i, 0)),
                  pl.BlockSpec((D, D), vec),
                  pl.BlockSpec((1, D), vec),
                  pl.BlockSpec((D, D), vec),
                  pl.BlockSpec((1, D), vec),
                  pl.BlockSpec((1, D), vec),
                  pl.BlockSpec((1, D), vec)],
        out_specs=pl.BlockSpec((bm, D), lambda i: (i, 0)),
        out_shape=jax.ShapeDtypeStruct((m, D), jnp.float32),
    )(m2g, gsum, w1a, b1, w2, b2, gm, bt)


def _node_mlp_body(a_ref, gn_ref, wa_ref, wg_ref, b1_ref, w2_ref, b2_ref,
                   gm_ref, bt_ref, o_ref):
    h = jnp.dot(a_ref[...], wa_ref[...],
                preferred_element_type=jnp.float32, precision=_HI)
    h = h + jnp.dot(gn_ref[...], wg_ref[...],
                    preferred_element_type=jnp.float32, precision=_HI)
    h = h + b1_ref[...]
    h = h * jax.nn.sigmoid(h)
    h = jnp.dot(h, w2_ref[...],
                preferred_element_type=jnp.float32, precision=_HI) + b2_ref[...]
    mu = jnp.mean(h, axis=-1, keepdims=True)
    var = jnp.mean((h - mu) ** 2, axis=-1, keepdims=True)
    h = (h - mu) * lax.rsqrt(var + 1e-5)
    o_ref[...] = h * gm_ref[...] + bt_ref[...] + gn_ref[...]


def _node_mlp(agg_pad, gn, wa, wg, b1, w2, b2, gm, bt, bm):
    vec = lambda i: (0, 0)
    return pl.pallas_call(
        _node_mlp_body,
        grid=(N_GRID // bm,),
        in_specs=[pl.BlockSpec((bm, D), lambda i: (i, 0)),
                  pl.BlockSpec((bm, D), lambda i: (i, 0)),
                  pl.BlockSpec((D, D), vec),
                  pl.BlockSpec((D, D), vec),
                  pl.BlockSpec((1, D), vec),
                  pl.BlockSpec((D, D), vec),
                  pl.BlockSpec((1, D), vec),
                  pl.BlockSpec((1, D), vec),
                  pl.BlockSpec((1, D), vec)],
        out_specs=pl.BlockSpec((bm, D), lambda i: (i, 0)),
        out_shape=jax.ShapeDtypeStruct((N_GRID, D), jnp.float32),
    )(agg_pad, gn, wa, wg, b1, w2, b2, gm, bt)


GW = 200
NWIN = N_GRID // GW
ECK = 1024
NTH = 640
RPB = 512


def _rowptr_body(d_ref, o_ref):
    i = pl.program_id(0)

    @pl.when(i == 0)
    def _():
        o_ref[...] = jnp.zeros_like(o_ref)

    d = d_ref[...].reshape(-1, 1)
    th = lax.broadcasted_iota(jnp.int32, (1, NTH), 1) * GW
    cnt = jnp.sum((d < th).astype(jnp.int32), axis=0, keepdims=True)
    o_ref[...] += cnt


def _tc_rowptr(dst):
    d2 = dst.reshape(N_EDGES // RPB, 1, RPB)
    return pl.pallas_call(
        _rowptr_body,
        grid=(N_EDGES // RPB,),
        in_specs=[pl.BlockSpec((1, 1, RPB), lambda i: (i, 0, 0))],
        out_specs=pl.BlockSpec((1, NTH), lambda i: (0, 0)),
        out_shape=jax.ShapeDtypeStruct((1, NTH), jnp.int32),
    )(d2)


def _segsum_body(rp_ref, ef_hbm, d_hbm, o_ref, acc,
                 ebuf0, ebuf1, dbuf0, dbuf1, es0, es1, ds0, ds1):
    g = pl.program_id(0)
    lo = rp_ref[0, g]
    hi = rp_ref[0, g + 1]
    gbase = g * GW
    wio = lax.broadcasted_iota(jnp.int32, (GW, 128), 0)
    lio = lax.broadcasted_iota(jnp.int32, (1, 128), 1)
    pos0 = (lo // 128) * 128
    acc[...] = jnp.zeros((GW, D), jnp.float32)
    ebufs, dbufs = (ebuf0, ebuf1), (dbuf0, dbuf1)
    esems, dsems = (es0, es1), (ds0, ds1)

    def start(pos, b):
        pos_r = jnp.minimum(pos, N_EDGES - ECK)
        pltpu.make_async_copy(
            ef_hbm.at[pl.ds(pos_r, ECK)], ebufs[b], esems[b]).start()
        pltpu.make_async_copy(
            d_hbm.at[pl.ds(pos_r // 128, ECK // 128)], dbufs[b], dsems[b]).start()

    @pl.when(pos0 < hi)
    def _():
        start(pos0, 0)

    def cond(state):
        return state[0] < hi

    def body(state):
        pos, it = state
        pos_r = jnp.minimum(pos, N_EDGES - ECK)
        nxt = pos_r + ECK

        def process(b):
            pltpu.make_async_copy(
                ef_hbm.at[pl.ds(pos_r, ECK)], ebufs[b], esems[b]).wait()
            pltpu.make_async_copy(
                d_hbm.at[pl.ds(pos_r // 128, ECK // 128)], dbufs[b],
                dsems[b]).wait()

            @pl.when(nxt < hi)
            def _():
                start(nxt, 1 - b)

            for h in range(ECK // 128):
                dv = dbufs[b][h, :].reshape(1, 128)
                ev = lio + (pos_r + h * 128)
                valid = (dv - gbase == wio) & (ev >= pos)
                oh = valid.astype(jnp.float32)
                acc[...] += jnp.dot(oh, ebufs[b][pl.ds(h * 128, 128), :],
                                    preferred_element_type=jnp.float32)

        @pl.when(it % 2 == 0)
        def _():
            process(0)

        @pl.when(it % 2 == 1)
        def _():
            process(1)

        return (nxt, it + 1)

    lax.while_loop(cond, body, (pos0, 0))
    o_ref[...] = acc[...]


def _tc_segment_sum(efeat, dst, rowptr):
    d2 = dst.reshape(N_EDGES // 128, 128)
    return pl.pallas_call(
        _segsum_body,
        grid=(NWIN,),
        in_specs=[pl.BlockSpec(memory_space=pltpu.SMEM),
                  pl.BlockSpec(memory_space=pltpu.HBM),
                  pl.BlockSpec(memory_space=pltpu.HBM)],
        out_specs=pl.BlockSpec((GW, D), lambda g: (g, 0)),
        out_shape=jax.ShapeDtypeStruct((N_GRID, D), jnp.float32),
        scratch_shapes=[pltpu.VMEM((GW, D), jnp.float32),
                        pltpu.VMEM((ECK, D), jnp.float32),
                        pltpu.VMEM((ECK, D), jnp.float32),
                        pltpu.VMEM((ECK // 128, 128), jnp.int32),
                        pltpu.VMEM((ECK // 128, 128), jnp.int32),
                        pltpu.SemaphoreType.DMA,
                        pltpu.SemaphoreType.DMA,
                        pltpu.SemaphoreType.DMA,
                        pltpu.SemaphoreType.DMA],
    )(rowptr, efeat, d2)


def _sc_gather_sum(meshp, gridp, src, dst):
    n_per_w = N_EDGES // NW
    ncheck = n_per_w // EC
    mesh_sc = plsc.VectorSubcoreMesh(core_axis_name="c", subcore_axis_name="s")

    @functools.partial(
        pl.kernel,
        mesh=mesh_sc,
        out_type=jax.ShapeDtypeStruct((N_EDGES, D), jnp.float32),
        scratch_types=[
            pltpu.VMEM((EC,), jnp.int32),
            pltpu.VMEM((EC,), jnp.int32),
            pltpu.VMEM((EC, D), jnp.float32),
            pltpu.VMEM((EC, D), jnp.float32),
            pltpu.VMEM((EC,), jnp.int32),
            pltpu.VMEM((EC,), jnp.int32),
            pltpu.VMEM((EC, D), jnp.float32),
            pltpu.VMEM((EC, D), jnp.float32),
            pltpu.SemaphoreType.DMA,
            pltpu.SemaphoreType.DMA,
            pltpu.SemaphoreType.DMA,
            pltpu.SemaphoreType.DMA,
        ],
    )
    def k(meshp_hbm, gridp_hbm, src_hbm, dst_hbm, out_hbm,
          i1a, i2a, r1a, r2a, i1b, i2b, r1b, r2b, s1a, s2a, s1b, s2b):
        wid = lax.axis_index("s") * NC + lax.axis_index("c")
        base = wid * n_per_w

        def fetch(pos, i1, i2, r1, r2, s1, s2):
            pltpu.sync_copy(src_hbm.at[pl.ds(pos, EC)], i1)
            pltpu.sync_copy(dst_hbm.at[pl.ds(pos, EC)], i2)
            pltpu.async_copy(meshp_hbm.at[i1], r1, s1)
            pltpu.async_copy(gridp_hbm.at[i2], r2, s2)

        def finish(pos, i1, i2, r1, r2, s1, s2):
            pltpu.make_async_copy(meshp_hbm.at[i1], r1, s1).wait()
            pltpu.make_async_copy(gridp_hbm.at[i2], r2, s2).wait()

            def addrow(r, c):
                for v in range(D // 16):
                    sl = pl.ds(v * 16, 16)
                    r1[r, sl] = r1[r, sl] + r2[r, sl]
                return c

            lax.fori_loop(0, EC, addrow, 0)
            pltpu.sync_copy(r1, out_hbm.at[pl.ds(pos, EC)])

        fetch(base, i1a, i2a, r1a, r2a, s1a, s2a)

        def pair(k2, carry):
            pa = base + (2 * k2) * EC
            fetch(pa + EC, i1b, i2b, r1b, r2b, s1b, s2b)
            finish(pa, i1a, i2a, r1a, r2a, s1a, s2a)
            fetch(pa + 2 * EC, i1a, i2a, r1a, r2a, s1a, s2a)
            finish(pa + EC, i1b, i2b, r1b, r2b, s1b, s2b)
            return carry

        lax.fori_loop(0, (ncheck - 1) // 2, pair, 0)
        finish(base + (ncheck - 1) * EC, i1a, i2a, r1a, r2a, s1a, s2a)

    return k(meshp, gridp, src, dst)


def kernel(m2g_efeat, grid_nfeat, mesh_nfeat, src, dst,
           eW1, eb1, eW2, eb2, eg, ebeta,
           nW1, nb1, nW2, nb2, ng, nbeta):
    src = src.astype(jnp.int32)
    dst = dst.astype(jnp.int32)
    w_e = eW1[:D]
    w_m = eW1[D:2 * D]
    w_g = eW1[2 * D:]
    mesh_proj = _rowblock_matmul(mesh_nfeat, w_m, bm=400)
    grid_proj = _rowblock_matmul(grid_nfeat, w_g, bm=1000)
    gsum = _sc_gather_sum(mesh_proj, grid_proj, src, dst)
    efeat = _edge_mlp(m2g_efeat, gsum, w_e,
                      eb1.reshape(1, D), eW2, eb2.reshape(1, D),
                      eg.reshape(1, D), ebeta.reshape(1, D), bm=512)
    rowptr = _tc_rowptr(dst)
    agg = _tc_segment_sum(efeat, dst, rowptr)
    out = _node_mlp(agg, grid_nfeat, nW1[:D], nW1[D:],
                    nb1.reshape(1, D), nW2, nb2.reshape(1, D),
                    ng.reshape(1, D), nbeta.reshape(1, D), bm=1000)
    return out


# speedup vs baseline: 2.2121x; 1.4195x over previous
import functools

import jax
import jax.numpy as jnp
from jax import lax
from jax.experimental import pallas as pl
from jax.experimental.pallas import tpu as pltpu
from jax.experimental.pallas import tpu_sc as plsc

N_MESH = 10000
N_GRID = 50000
N_EDGES = 320000
D = 128

NC = 2
NS = 16
NW = NC * NS

G_CHUNK = 12800
N_CHUNKS = 4
G_PAD = G_CHUNK * N_CHUNKS
EC = 80

_HI = lax.Precision.HIGHEST


def _matmul_body(x_ref, w_ref, o_ref):
    o_ref[...] = jnp.dot(x_ref[...], w_ref[...],
                         preferred_element_type=jnp.float32)


def _rowblock_matmul(x, w, bm):
    m, k = x.shape
    n = w.shape[1]
    return pl.pallas_call(
        _matmul_body,
        grid=(m // bm,),
        in_specs=[pl.BlockSpec((bm, k), lambda i: (i, 0)),
                  pl.BlockSpec((k, n), lambda i: (0, 0))],
        out_specs=pl.BlockSpec((bm, n), lambda i: (i, 0)),
        out_shape=jax.ShapeDtypeStruct((m, n), jnp.float32),
    )(x, w)


def _edge_mlp_body(x_ref, g_ref, w1_ref, b1_ref, w2_ref, b2_ref,
                   gm_ref, bt_ref, o_ref):
    h = jnp.dot(x_ref[...], w1_ref[...],
                preferred_element_type=jnp.float32)
    h = h + g_ref[...] + b1_ref[...]
    h = h * jax.nn.sigmoid(h)
    h = jnp.dot(h, w2_ref[...],
                preferred_element_type=jnp.float32) + b2_ref[...]
    mu = jnp.mean(h, axis=-1, keepdims=True)
    var = jnp.mean((h - mu) ** 2, axis=-1, keepdims=True)
    h = (h - mu) * lax.rsqrt(var + 1e-5)
    o_ref[...] = h * gm_ref[...] + bt_ref[...]


def _edge_mlp(m2g, gsum, w1a, b1, w2, b2, gm, bt, bm):
    m = m2g.shape[0]
    vec = lambda i: (0, 0)
    return pl.pallas_call(
        _edge_mlp_body,
        grid=(m // bm,),
        in_specs=[pl.BlockSpec((bm, D), lambda i: (i, 0)),
                  pl.BlockSpec((bm, D), lambda i: (i, 0)),
                  pl.BlockSpec((D, D), vec),
                  pl.BlockSpec((1, D), vec),
                  pl.BlockSpec((D, D), vec),
                  pl.BlockSpec((1, D), vec),
                  pl.BlockSpec((1, D), vec),
                  pl.BlockSpec((1, D), vec)],
        out_specs=pl.BlockSpec((bm, D), lambda i: (i, 0)),
        out_shape=jax.ShapeDtypeStruct((m, D), jnp.float32),
    )(m2g, gsum, w1a, b1, w2, b2, gm, bt)


def _node_mlp_body(a_ref, gn_ref, wa_ref, wg_ref, b1_ref, w2_ref, b2_ref,
                   gm_ref, bt_ref, o_ref):
    h = jnp.dot(a_ref[...], wa_ref[...],
                preferred_element_type=jnp.float32)
    h = h + jnp.dot(gn_ref[...], wg_ref[...],
                    preferred_element_type=jnp.float32)
    h = h + b1_ref[...]
    h = h * jax.nn.sigmoid(h)
    h = jnp.dot(h, w2_ref[...],
                preferred_element_type=jnp.float32) + b2_ref[...]
    mu = jnp.mean(h, axis=-1, keepdims=True)
    var = jnp.mean((h - mu) ** 2, axis=-1, keepdims=True)
    h = (h - mu) * lax.rsqrt(var + 1e-5)
    o_ref[...] = h * gm_ref[...] + bt_ref[...] + gn_ref[...]


def _node_mlp(agg_pad, gn, wa, wg, b1, w2, b2, gm, bt, bm):
    vec = lambda i: (0, 0)
    return pl.pallas_call(
        _node_mlp_body,
        grid=(N_GRID // bm,),
        in_specs=[pl.BlockSpec((bm, D), lambda i: (i, 0)),
                  pl.BlockSpec((bm, D), lambda i: (i, 0)),
                  pl.BlockSpec((D, D), vec),
                  pl.BlockSpec((D, D), vec),
                  pl.BlockSpec((1, D), vec),
                  pl.BlockSpec((D, D), vec),
                  pl.BlockSpec((1, D), vec),
                  pl.BlockSpec((1, D), vec),
                  pl.BlockSpec((1, D), vec)],
        out_specs=pl.BlockSpec((bm, D), lambda i: (i, 0)),
        out_shape=jax.ShapeDtypeStruct((N_GRID, D), jnp.float32),
    )(agg_pad, gn, wa, wg, b1, w2, b2, gm, bt)


GW = 400
NWIN = N_GRID // GW
ECK = 1024
NTH = 640
RPB = 512


def _rowptr_body(d_ref, o_ref):
    i = pl.program_id(0)

    @pl.when(i == 0)
    def _():
        o_ref[...] = jnp.zeros_like(o_ref)

    th = lax.broadcasted_iota(jnp.int32, (1, NTH), 1) * GW
    cnt = jnp.zeros((1, NTH), jnp.int32)
    for j in range(5):
        d = d_ref[j].reshape(RPB, 1)
        cnt += jnp.sum((d < th).astype(jnp.int32), axis=0, keepdims=True)
    o_ref[...] += cnt


def _tc_rowptr(dst):
    d2 = dst.reshape(N_EDGES // RPB, 1, RPB)
    return pl.pallas_call(
        _rowptr_body,
        grid=(N_EDGES // (5 * RPB),),
        in_specs=[pl.BlockSpec((5, 1, RPB), lambda i: (i, 0, 0))],
        out_specs=pl.BlockSpec((1, NTH), lambda i: (0, 0)),
        out_shape=jax.ShapeDtypeStruct((1, NTH), jnp.int32),
    )(d2)


def _segsum_body(rp_ref, ef_hbm, d_hbm, o_ref, acc,
                 ebuf0, ebuf1, dbuf0, dbuf1, es0, es1, ds0, ds1):
    g = pl.program_id(0)
    lo = rp_ref[0, g]
    hi = rp_ref[0, g + 1]
    gbase = g * GW
    wio = lax.broadcasted_iota(jnp.int32, (GW, 128), 0)
    lio = lax.broadcasted_iota(jnp.int32, (1, 128), 1)
    pos0 = (lo // 128) * 128
    acc[...] = jnp.zeros((GW, D), jnp.float32)
    ebufs, dbufs = (ebuf0, ebuf1), (dbuf0, dbuf1)
    esems, dsems = (es0, es1), (ds0, ds1)

    def start(pos, b):
        pos_r = jnp.minimum(pos, N_EDGES - ECK)
        pltpu.make_async_copy(
            ef_hbm.at[pl.ds(pos_r, ECK)], ebufs[b], esems[b]).start()
        pltpu.make_async_copy(
            d_hbm.at[pl.ds(pos_r // 128, ECK // 128)], dbufs[b], dsems[b]).start()

    @pl.when(pos0 < hi)
    def _():
        start(pos0, 0)

    def cond(state):
        return state[0] < hi

    def body(state):
        pos, it = state
        pos_r = jnp.minimum(pos, N_EDGES - ECK)
        nxt = pos_r + ECK

        def process(b):
            pltpu.make_async_copy(
                ef_hbm.at[pl.ds(pos_r, ECK)], ebufs[b], esems[b]).wait()
            pltpu.make_async_copy(
                d_hbm.at[pl.ds(pos_r // 128, ECK // 128)], dbufs[b],
                dsems[b]).wait()

            @pl.when(nxt < hi)
            def _():
                start(nxt, 1 - b)

            for h in range(ECK // 128):
                dv = dbufs[b][h, :].reshape(1, 128)
                ev = lio + (pos_r + h * 128)
                valid = (dv - gbase == wio) & (ev >= pos)
                oh = valid.astype(jnp.float32)
                acc[...] += jnp.dot(oh, ebufs[b][pl.ds(h * 128, 128), :],
                                    preferred_element_type=jnp.float32)

        @pl.when(it % 2 == 0)
        def _():
            process(0)

        @pl.when(it % 2 == 1)
        def _():
            process(1)

        return (nxt, it + 1)

    lax.while_loop(cond, body, (pos0, 0))
    o_ref[...] = acc[...]


def _tc_segment_sum(efeat, dst, rowptr):
    d2 = dst.reshape(N_EDGES // 128, 128)
    return pl.pallas_call(
        _segsum_body,
        grid=(NWIN,),
        in_specs=[pl.BlockSpec(memory_space=pltpu.SMEM),
                  pl.BlockSpec(memory_space=pltpu.HBM),
                  pl.BlockSpec(memory_space=pltpu.HBM)],
        out_specs=pl.BlockSpec((GW, D), lambda g: (g, 0)),
        out_shape=jax.ShapeDtypeStruct((N_GRID, D), jnp.float32),
        scratch_shapes=[pltpu.VMEM((GW, D), jnp.float32),
                        pltpu.VMEM((ECK, D), jnp.float32),
                        pltpu.VMEM((ECK, D), jnp.float32),
                        pltpu.VMEM((ECK // 128, 128), jnp.int32),
                        pltpu.VMEM((ECK // 128, 128), jnp.int32),
                        pltpu.SemaphoreType.DMA,
                        pltpu.SemaphoreType.DMA,
                        pltpu.SemaphoreType.DMA,
                        pltpu.SemaphoreType.DMA],
    )(rowptr, efeat, d2)


def _sc_gather_sum(meshp, gridp, src, dst):
    n_per_w = N_EDGES // NW
    ncheck = n_per_w // EC
    mesh_sc = plsc.VectorSubcoreMesh(core_axis_name="c", subcore_axis_name="s")

    @functools.partial(
        pl.kernel,
        mesh=mesh_sc,
        out_type=jax.ShapeDtypeStruct((N_EDGES, D), jnp.float32),
        scratch_types=[
            pltpu.VMEM((EC,), jnp.int32),
            pltpu.VMEM((EC,), jnp.int32),
            pltpu.VMEM((EC, D), jnp.float32),
            pltpu.VMEM((EC, D), jnp.float32),
            pltpu.VMEM((EC,), jnp.int32),
            pltpu.VMEM((EC,), jnp.int32),
            pltpu.VMEM((EC, D), jnp.float32),
            pltpu.VMEM((EC, D), jnp.float32),
            pltpu.SemaphoreType.DMA,
            pltpu.SemaphoreType.DMA,
            pltpu.SemaphoreType.DMA,
            pltpu.SemaphoreType.DMA,
        ],
    )
    def k(meshp_hbm, gridp_hbm, src_hbm, dst_hbm, out_hbm,
          i1a, i2a, r1a, r2a, i1b, i2b, r1b, r2b, s1a, s2a, s1b, s2b):
        wid = lax.axis_index("s") * NC + lax.axis_index("c")
        base = wid * n_per_w

        def fetch(pos, i1, i2, r1, r2, s1, s2):
            pltpu.sync_copy(src_hbm.at[pl.ds(pos, EC)], i1)
            pltpu.sync_copy(dst_hbm.at[pl.ds(pos, EC)], i2)
            pltpu.async_copy(meshp_hbm.at[i1], r1, s1)
            pltpu.async_copy(gridp_hbm.at[i2], r2, s2)

        def finish(pos, i1, i2, r1, r2, s1, s2):
            pltpu.make_async_copy(meshp_hbm.at[i1], r1, s1).wait()
            pltpu.make_async_copy(gridp_hbm.at[i2], r2, s2).wait()

            def addrow(r, c):
                for v in range(D // 16):
                    sl = pl.ds(v * 16, 16)
                    r1[r, sl] = r1[r, sl] + r2[r, sl]
                return c

            lax.fori_loop(0, EC, addrow, 0)
            pltpu.sync_copy(r1, out_hbm.at[pl.ds(pos, EC)])

        fetch(base, i1a, i2a, r1a, r2a, s1a, s2a)

        def pair(k2, carry):
            pa = base + (2 * k2) * EC
            fetch(pa + EC, i1b, i2b, r1b, r2b, s1b, s2b)
            finish(pa, i1a, i2a, r1a, r2a, s1a, s2a)
            fetch(pa + 2 * EC, i1a, i2a, r1a, r2a, s1a, s2a)
            finish(pa + EC, i1b, i2b, r1b, r2b, s1b, s2b)
            return carry

        lax.fori_loop(0, (ncheck - 1) // 2, pair, 0)
        finish(base + (ncheck - 1) * EC, i1a, i2a, r1a, r2a, s1a, s2a)

    return k(meshp, gridp, src, dst)


def kernel(m2g_efeat, grid_nfeat, mesh_nfeat, src, dst,
           eW1, eb1, eW2, eb2, eg, ebeta,
           nW1, nb1, nW2, nb2, ng, nbeta):
    src = src.astype(jnp.int32)
    dst = dst.astype(jnp.int32)
    w_e = eW1[:D]
    w_m = eW1[D:2 * D]
    w_g = eW1[2 * D:]
    mesh_proj = _rowblock_matmul(mesh_nfeat, w_m, bm=400)
    grid_proj = _rowblock_matmul(grid_nfeat, w_g, bm=1000)
    gsum = _sc_gather_sum(mesh_proj, grid_proj, src, dst)
    efeat = _edge_mlp(m2g_efeat, gsum, w_e,
                      eb1.reshape(1, D), eW2, eb2.reshape(1, D),
                      eg.reshape(1, D), ebeta.reshape(1, D), bm=512)
    rowptr = _tc_rowptr(dst)
    agg = _tc_segment_sum(efeat, dst, rowptr)
    out = _node_mlp(agg, grid_nfeat, nW1[:D], nW1[D:],
                    nb1.reshape(1, D), nW2, nb2.reshape(1, D),
                    ng.reshape(1, D), nbeta.reshape(1, D), bm=1000)
    return out


# speedup vs baseline: 2.2126x; 1.0002x over previous
import functools

import jax
import jax.numpy as jnp
from jax import lax
from jax.experimental import pallas as pl
from jax.experimental.pallas import tpu as pltpu
from jax.experimental.pallas import tpu_sc as plsc

N_MESH = 10000
N_GRID = 50000
N_EDGES = 320000
D = 128

NC = 2
NS = 16
NW = NC * NS

EC = 80


def _matmul_body(x_ref, w_ref, o_ref):
    o_ref[...] = jnp.dot(x_ref[...], w_ref[...],
                         preferred_element_type=jnp.float32)


def _rowblock_matmul(x, w, bm):
    m, k = x.shape
    n = w.shape[1]
    return pl.pallas_call(
        _matmul_body,
        grid=(m // bm,),
        in_specs=[pl.BlockSpec((bm, k), lambda i: (i, 0)),
                  pl.BlockSpec((k, n), lambda i: (0, 0))],
        out_specs=pl.BlockSpec((bm, n), lambda i: (i, 0)),
        out_shape=jax.ShapeDtypeStruct((m, n), jnp.float32),
    )(x, w)


def _edge_mlp_body(x_ref, g_ref, w1_ref, b1_ref, w2_ref, b2_ref,
                   gm_ref, bt_ref, o_ref):
    h = jnp.dot(x_ref[...], w1_ref[...],
                preferred_element_type=jnp.float32)
    h = h + g_ref[...] + b1_ref[...]
    h = h * jax.nn.sigmoid(h)
    h = jnp.dot(h, w2_ref[...],
                preferred_element_type=jnp.float32) + b2_ref[...]
    mu = jnp.mean(h, axis=-1, keepdims=True)
    var = jnp.mean((h - mu) ** 2, axis=-1, keepdims=True)
    h = (h - mu) * lax.rsqrt(var + 1e-5)
    o_ref[...] = h * gm_ref[...] + bt_ref[...]


def _edge_mlp(m2g, gsum, w1a, b1, w2, b2, gm, bt, bm):
    m = m2g.shape[0]
    vec = lambda i: (0, 0)
    return pl.pallas_call(
        _edge_mlp_body,
        grid=(m // bm,),
        in_specs=[pl.BlockSpec((bm, D), lambda i: (i, 0)),
                  pl.BlockSpec((bm, D), lambda i: (i, 0)),
                  pl.BlockSpec((D, D), vec),
                  pl.BlockSpec((1, D), vec),
                  pl.BlockSpec((D, D), vec),
                  pl.BlockSpec((1, D), vec),
                  pl.BlockSpec((1, D), vec),
                  pl.BlockSpec((1, D), vec)],
        out_specs=pl.BlockSpec((bm, D), lambda i: (i, 0)),
        out_shape=jax.ShapeDtypeStruct((m, D), jnp.float32),
    )(m2g, gsum, w1a, b1, w2, b2, gm, bt)


def _node_mlp_body(a_ref, gn_ref, wa_ref, wg_ref, b1_ref, w2_ref, b2_ref,
                   gm_ref, bt_ref, o_ref):
    h = jnp.dot(a_ref[...], wa_ref[...],
                preferred_element_type=jnp.float32)
    h = h + jnp.dot(gn_ref[...], wg_ref[...],
                    preferred_element_type=jnp.float32)
    h = h + b1_ref[...]
    h = h * jax.nn.sigmoid(h)
    h = jnp.dot(h, w2_ref[...],
                preferred_element_type=jnp.float32) + b2_ref[...]
    mu = jnp.mean(h, axis=-1, keepdims=True)
    var = jnp.mean((h - mu) ** 2, axis=-1, keepdims=True)
    h = (h - mu) * lax.rsqrt(var + 1e-5)
    o_ref[...] = h * gm_ref[...] + bt_ref[...] + gn_ref[...]


def _node_mlp(agg_pad, gn, wa, wg, b1, w2, b2, gm, bt, bm):
    vec = lambda i: (0, 0)
    return pl.pallas_call(
        _node_mlp_body,
        grid=(N_GRID // bm,),
        in_specs=[pl.BlockSpec((bm, D), lambda i: (i, 0)),
                  pl.BlockSpec((bm, D), lambda i: (i, 0)),
                  pl.BlockSpec((D, D), vec),
                  pl.BlockSpec((D, D), vec),
                  pl.BlockSpec((1, D), vec),
                  pl.BlockSpec((D, D), vec),
                  pl.BlockSpec((1, D), vec),
                  pl.BlockSpec((1, D), vec),
                  pl.BlockSpec((1, D), vec)],
        out_specs=pl.BlockSpec((bm, D), lambda i: (i, 0)),
        out_shape=jax.ShapeDtypeStruct((N_GRID, D), jnp.float32),
    )(agg_pad, gn, wa, wg, b1, w2, b2, gm, bt)


GW = 400
NWIN = N_GRID // GW
ECK = 1024
NTH = 640
RPB = 512


def _rowptr_body(d_ref, o_ref):
    i = pl.program_id(0)

    @pl.when(i == 0)
    def _():
        o_ref[...] = jnp.zeros_like(o_ref)

    th = lax.broadcasted_iota(jnp.int32, (1, NTH), 1) * GW
    cnt = jnp.zeros((1, NTH), jnp.int32)
    for j in range(5):
        d = d_ref[j].reshape(RPB, 1)
        cnt += jnp.sum((d < th).astype(jnp.int32), axis=0, keepdims=True)
    o_ref[...] += cnt


def _tc_rowptr(dst):
    d2 = dst.reshape(N_EDGES // RPB, 1, RPB)
    return pl.pallas_call(
        _rowptr_body,
        grid=(N_EDGES // (5 * RPB),),
        in_specs=[pl.BlockSpec((5, 1, RPB), lambda i: (i, 0, 0))],
        out_specs=pl.BlockSpec((1, NTH), lambda i: (0, 0)),
        out_shape=jax.ShapeDtypeStruct((1, NTH), jnp.int32),
    )(d2)


def _segsum_body(rp_ref, ef_hbm, d_hbm, o_ref, acc,
                 ebuf0, ebuf1, dbuf0, dbuf1, es0, es1, ds0, ds1):
    g = pl.program_id(0)
    lo = rp_ref[0, g]
    hi = rp_ref[0, g + 1]
    gbase = g * GW
    wio = lax.broadcasted_iota(jnp.int32, (GW, 128), 0)
    lio = lax.broadcasted_iota(jnp.int32, (1, 128), 1)
    pos0 = (lo // 128) * 128
    acc[...] = jnp.zeros((GW, D), jnp.float32)
    ebufs, dbufs = (ebuf0, ebuf1), (dbuf0, dbuf1)
    esems, dsems = (es0, es1), (ds0, ds1)

    def start(pos, b):
        pos_r = jnp.minimum(pos, N_EDGES - ECK)
        pltpu.make_async_copy(
            ef_hbm.at[pl.ds(pos_r, ECK)], ebufs[b], esems[b]).start()
        pltpu.make_async_copy(
            d_hbm.at[pl.ds(pos_r // 128, ECK // 128)], dbufs[b], dsems[b]).start()

    @pl.when(pos0 < hi)
    def _():
        start(pos0, 0)

    def cond(state):
        return state[0] < hi

    def body(state):
        pos, it = state
        pos_r = jnp.minimum(pos, N_EDGES - ECK)
        nxt = pos_r + ECK

        def process(b):
            pltpu.make_async_copy(
                ef_hbm.at[pl.ds(pos_r, ECK)], ebufs[b], esems[b]).wait()
            pltpu.make_async_copy(
                d_hbm.at[pl.ds(pos_r // 128, ECK // 128)], dbufs[b],
                dsems[b]).wait()

            @pl.when(nxt < hi)
            def _():
                start(nxt, 1 - b)

            for h in range(ECK // 128):
                dv = dbufs[b][h, :].reshape(1, 128)
                ev = lio + (pos_r + h * 128)
                valid = (dv - gbase == wio) & (ev >= pos)
                oh = valid.astype(jnp.float32)
                acc[...] += jnp.dot(oh, ebufs[b][pl.ds(h * 128, 128), :],
                                    preferred_element_type=jnp.float32)

        @pl.when(it % 2 == 0)
        def _():
            process(0)

        @pl.when(it % 2 == 1)
        def _():
            process(1)

        return (nxt, it + 1)

    lax.while_loop(cond, body, (pos0, 0))
    o_ref[...] = acc[...]


def _tc_segment_sum(efeat, dst, rowptr):
    d2 = dst.reshape(N_EDGES // 128, 128)
    return pl.pallas_call(
        _segsum_body,
        grid=(NWIN,),
        in_specs=[pl.BlockSpec(memory_space=pltpu.SMEM),
                  pl.BlockSpec(memory_space=pltpu.HBM),
                  pl.BlockSpec(memory_space=pltpu.HBM)],
        out_specs=pl.BlockSpec((GW, D), lambda g: (g, 0)),
        out_shape=jax.ShapeDtypeStruct((N_GRID, D), jnp.float32),
        scratch_shapes=[pltpu.VMEM((GW, D), jnp.float32),
                        pltpu.VMEM((ECK, D), jnp.float32),
                        pltpu.VMEM((ECK, D), jnp.float32),
                        pltpu.VMEM((ECK // 128, 128), jnp.int32),
                        pltpu.VMEM((ECK // 128, 128), jnp.int32),
                        pltpu.SemaphoreType.DMA,
                        pltpu.SemaphoreType.DMA,
                        pltpu.SemaphoreType.DMA,
                        pltpu.SemaphoreType.DMA],
    )(rowptr, efeat, d2)


def _sc_gather_sum(meshp, gridp, src, dst):
    n_per_w = N_EDGES // NW
    ncheck = n_per_w // EC
    mesh_sc = plsc.VectorSubcoreMesh(core_axis_name="c", subcore_axis_name="s")

    @functools.partial(
        pl.kernel,
        mesh=mesh_sc,
        out_type=jax.ShapeDtypeStruct((N_EDGES, D), jnp.float32),
        scratch_types=[
            pltpu.VMEM((EC,), jnp.int32),
            pltpu.VMEM((EC,), jnp.int32),
            pltpu.VMEM((EC, D), jnp.float32),
            pltpu.VMEM((EC, D), jnp.float32),
            pltpu.VMEM((EC,), jnp.int32),
            pltpu.VMEM((EC,), jnp.int32),
            pltpu.VMEM((EC, D), jnp.float32),
            pltpu.VMEM((EC, D), jnp.float32),
            pltpu.SemaphoreType.DMA,
            pltpu.SemaphoreType.DMA,
            pltpu.SemaphoreType.DMA,
            pltpu.SemaphoreType.DMA,
        ],
    )
    def k(meshp_hbm, gridp_hbm, src_hbm, dst_hbm, out_hbm,
          i1a, i2a, r1a, r2a, i1b, i2b, r1b, r2b, s1a, s2a, s1b, s2b):
        wid = lax.axis_index("s") * NC + lax.axis_index("c")
        base = wid * n_per_w

        def fetch(pos, i1, i2, r1, r2, s1, s2):
            pltpu.sync_copy(src_hbm.at[pl.ds(pos, EC)], i1)
            pltpu.sync_copy(dst_hbm.at[pl.ds(pos, EC)], i2)
            pltpu.async_copy(meshp_hbm.at[i1], r1, s1)
            pltpu.async_copy(gridp_hbm.at[i2], r2, s2)

        def finish(pos, i1, i2, r1, r2, s1, s2):
            pltpu.make_async_copy(meshp_hbm.at[i1], r1, s1).wait()
            pltpu.make_async_copy(gridp_hbm.at[i2], r2, s2).wait()

            def addrow(r, c):
                for v in range(D // 16):
                    sl = pl.ds(v * 16, 16)
                    r1[r, sl] = r1[r, sl] + r2[r, sl]
                return c

            lax.fori_loop(0, EC, addrow, 0)
            pltpu.sync_copy(r1, out_hbm.at[pl.ds(pos, EC)])

        fetch(base, i1a, i2a, r1a, r2a, s1a, s2a)

        def pair(k2, carry):
            pa = base + (2 * k2) * EC
            fetch(pa + EC, i1b, i2b, r1b, r2b, s1b, s2b)
            finish(pa, i1a, i2a, r1a, r2a, s1a, s2a)
            fetch(pa + 2 * EC, i1a, i2a, r1a, r2a, s1a, s2a)
            finish(pa + EC, i1b, i2b, r1b, r2b, s1b, s2b)
            return carry

        lax.fori_loop(0, (ncheck - 1) // 2, pair, 0)
        finish(base + (ncheck - 1) * EC, i1a, i2a, r1a, r2a, s1a, s2a)

    return k(meshp, gridp, src, dst)


def kernel(m2g_efeat, grid_nfeat, mesh_nfeat, src, dst,
           eW1, eb1, eW2, eb2, eg, ebeta,
           nW1, nb1, nW2, nb2, ng, nbeta):
    src = src.astype(jnp.int32)
    dst = dst.astype(jnp.int32)
    w_e = eW1[:D]
    w_m = eW1[D:2 * D]
    w_g = eW1[2 * D:]
    mesh_proj = _rowblock_matmul(mesh_nfeat, w_m, bm=400)
    grid_proj = _rowblock_matmul(grid_nfeat, w_g, bm=1000)
    gsum = _sc_gather_sum(mesh_proj, grid_proj, src, dst)
    efeat = _edge_mlp(m2g_efeat, gsum, w_e,
                      eb1.reshape(1, D), eW2, eb2.reshape(1, D),
                      eg.reshape(1, D), ebeta.reshape(1, D), bm=512)
    rowptr = _tc_rowptr(dst)
    agg = _tc_segment_sum(efeat, dst, rowptr)
    out = _node_mlp(agg, grid_nfeat, nW1[:D], nW1[D:],
                    nb1.reshape(1, D), nW2, nb2.reshape(1, D),
                    ng.reshape(1, D), nbeta.reshape(1, D), bm=1000)
    return out
